# baseline (speedup 1.0000x reference)
import functools

import jax
import jax.numpy as jnp
from jax import lax
from jax.experimental import pallas as pl
from jax.experimental.pallas import tpu as pltpu
from jax.experimental.pallas import tpu_sc as plsc

N = 10000
E = 320000
D = 128
C = 64
L = 24
NCLS = 10

NC = 2
NS = 16
NW = NC * NS
LN = 16

NP = 10240
RPT = NP // NS
CH = 128
ECHUNKS = E // CH
NCH_W = 80
EPC = NCH_W * NW
EP = EPC * CH
NODECH = RPT // CH
NB = 6
LA = 3

_mesh = plsc.VectorSubcoreMesh(
    core_axis_name="c", subcore_axis_name="s", num_cores=NC, num_subcores=NS)


@functools.partial(
    pl.kernel,
    out_type=jax.ShapeDtypeStruct((NC, NP), jnp.float32),
    mesh=_mesh,
    scratch_types=[
        pltpu.VMEM_SHARED((NP,), jnp.float32),
        pltpu.VMEM((NCH_W, CH), jnp.int32),
        pltpu.VMEM((CH,), jnp.float32),
        pltpu.VMEM((RPT,), jnp.float32),
    ],
    compiler_params=pltpu.CompilerParams(use_tc_tiling_on_sc=False),
)
def _sc_degree(dst_hbm, out_hbm, acc, didx, ones_v, slice_v):
    cid = lax.axis_index("c")
    sid = lax.axis_index("s")
    wid = sid * NC + cid
    base = sid * RPT

    def _z(i, carry):
        slice_v[pl.ds(i * LN, LN)] = jnp.zeros((LN,), jnp.float32)
        return carry
    lax.fori_loop(0, RPT // LN, _z, 0)
    pltpu.sync_copy(slice_v, acc.at[pl.ds(base, RPT)])

    def _o(i, carry):
        ones_v[pl.ds(i * LN, LN)] = jnp.ones((LN,), jnp.float32)
        return carry
    lax.fori_loop(0, CH // LN, _o, 0)
    pltpu.sync_copy(dst_hbm.at[pl.ds(wid * NCH_W, NCH_W)], didx)
    plsc.subcore_barrier()

    def _body(i, carry):
        pltpu.sync_copy(ones_v, acc.at[didx.at[i]], add=True)
        return carry
    lax.fori_loop(0, NCH_W, _body, 0)
    plsc.subcore_barrier()

    pltpu.sync_copy(acc.at[pl.ds(base, RPT)], slice_v)
    pltpu.sync_copy(slice_v, out_hbm.at[cid, pl.ds(base, RPT)])


@functools.partial(
    pl.kernel,
    out_type=jax.ShapeDtypeStruct((NC, NP, C), jnp.float32),
    mesh=_mesh,
    scratch_types=[
        pltpu.VMEM_SHARED((NP, C), jnp.float32),
        pltpu.VMEM((NCH_W * CH,), jnp.int32),
        pltpu.VMEM((NCH_W, CH), jnp.int32),
        pltpu.VMEM((NCH_W, CH), jnp.float32),
        pltpu.VMEM((NB, CH, C), jnp.float32),
        pltpu.VMEM((RPT,), jnp.float32),
        pltpu.SemaphoreType.DMA((NB,)),
        pltpu.SemaphoreType.DMA((NB,)),
    ],
    compiler_params=pltpu.CompilerParams(use_tc_tiling_on_sc=False),
)
def _sc_aggregate(h_hbm, srcl, dstl, srcg, dstg, wg_hbm, invd_hbm,
                  out_hbm, acc, srcb, dstb, wb, rows, invd,
                  gsem, ssem):
    cid = lax.axis_index("c")
    sid = lax.axis_index("s")
    wid = sid * NC + cid
    base = sid * RPT
    cbase = wid * NCH_W

    def _z(r, carry):
        for j in range(C // LN):
            rows[0, r, pl.ds(j * LN, LN)] = jnp.zeros((LN,), jnp.float32)
        return carry
    lax.fori_loop(0, CH, _z, 0)
    for k in range(NODECH):
        pltpu.sync_copy(rows.at[0], acc.at[pl.ds(base + k * CH, CH)])
    plsc.subcore_barrier()

    def _edge_loop(src_flat, dst2d, weighted):
        pltpu.sync_copy(src_flat.at[pl.ds(cbase * CH, NCH_W * CH)], srcb)
        pltpu.sync_copy(dst2d.at[pl.ds(cbase, NCH_W)], dstb)
        if weighted:
            pltpu.sync_copy(wg_hbm.at[pl.ds(cbase, NCH_W)], wb)

        ngroups = (NCH_W + LA + NB - 1) // NB + 1

        def _grp(g, carry):
            for u in range(NB):
                i = g * NB + u

                @pl.when(i < NCH_W)
                def _issue():
                    @pl.when(i >= NB)
                    def _drain_prev():
                        pltpu.make_async_copy(
                            rows.at[u], acc.at[dstb.at[0]], ssem.at[u]).wait()
                    pltpu.async_copy(
                        h_hbm.at[srcb.at[pl.ds(i * CH, CH)]],
                        rows.at[u], gsem.at[u])

                k = i - LA
                bu = (u + LA) % NB

                @pl.when(jnp.logical_and(k >= 0, k < NCH_W))
                def _process():
                    pltpu.make_async_copy(
                        h_hbm.at[srcb.at[pl.ds(0, CH)]],
                        rows.at[bu], gsem.at[bu]).wait()
                    if weighted:
                        def _scale(kk, c3):
                            w16 = wb[k, pl.ds(kk * LN, LN)]
                            for ii in range(LN):
                                w = w16[ii]
                                r = kk * LN + ii
                                for j in range(C // LN):
                                    rows[bu, r, pl.ds(j * LN, LN)] = (
                                        rows[bu, r, pl.ds(j * LN, LN)] * w)
                            return c3
                        lax.fori_loop(0, CH // LN, _scale, 0)
                    pltpu.async_copy(
                        rows.at[bu], acc.at[dstb.at[k]], ssem.at[bu],
                        add=True)
            return carry
        lax.fori_loop(0, ngroups, _grp, 0)

        for u in range(NB):
            pltpu.make_async_copy(
                rows.at[u], acc.at[dstb.at[0]], ssem.at[u]).wait()

    _edge_loop(srcl, dstl, weighted=False)
    plsc.subcore_barrier()

    pltpu.sync_copy(invd_hbm.at[pl.ds(base, RPT)], invd)
    for k in range(NODECH):
        rb = base + k * CH
        pltpu.sync_copy(acc.at[pl.ds(rb, CH)], rows.at[0])

        def _isc(k2, carry):
            s16 = invd[pl.ds(k * CH + k2 * LN, LN)]
            for i in range(LN):
                r = k2 * LN + i
                s = s16[i]
                for j in range(C // LN):
                    rows[0, r, pl.ds(j * LN, LN)] = (
                        rows[0, r, pl.ds(j * LN, LN)] * s)
            return carry
        lax.fori_loop(0, CH // LN, _isc, 0)
        pltpu.sync_copy(rows.at[0], acc.at[pl.ds(rb, CH)])
    plsc.subcore_barrier()

    _edge_loop(srcg, dstg, weighted=True)
    plsc.subcore_barrier()

    for k in range(NODECH):
        rb = base + k * CH
        pltpu.sync_copy(acc.at[pl.ds(rb, CH)], rows.at[0])
        pltpu.sync_copy(rows.at[0], out_hbm.at[cid, pl.ds(rb, CH)])


def _pre_body(x_ref, w_ref, b_ref, deg_ref, ewg_ref, sc_ref,
              h0_ref, ax0_ref, invd_ref, wg_ref):
    alpha = sc_ref[0, 0]
    gamma = sc_ref[0, 1]
    a1 = 1.0 - alpha
    h0 = jnp.dot(x_ref[...], w_ref[...],
                 preferred_element_type=jnp.float32) + b_ref[...]
    h0_ref[...] = h0
    ax0_ref[...] = alpha * h0
    d = deg_ref[0] + deg_ref[1]
    invd_ref[...] = a1 / jnp.maximum(d, 1.0)
    wg_ref[...] = (a1 * gamma) * ewg_ref[...]


_tc_pre = pl.pallas_call(
    _pre_body,
    out_shape=(
        jax.ShapeDtypeStruct((NP, C), jnp.float32),
        jax.ShapeDtypeStruct((NP, C), jnp.float32),
        jax.ShapeDtypeStruct((NP // 128, 128), jnp.float32),
        jax.ShapeDtypeStruct((ECHUNKS, 128), jnp.float32),
    ),
    in_specs=[
        pl.BlockSpec(memory_space=pltpu.VMEM),
        pl.BlockSpec(memory_space=pltpu.VMEM),
        pl.BlockSpec(memory_space=pltpu.VMEM),
        pl.BlockSpec(memory_space=pltpu.VMEM),
        pl.BlockSpec(memory_space=pltpu.VMEM),
        pl.BlockSpec(memory_space=pltpu.SMEM),
    ],
)


def _layer_body(p_ref, ax0_ref, w_ref, b_ref, h_ref):
    hp = p_ref[0] + p_ref[1] + ax0_ref[...]
    h = jnp.dot(hp, w_ref[...], preferred_element_type=jnp.float32) + b_ref[...]
    h_ref[...] = jnp.maximum(h, 0.0)


_tc_layer = pl.pallas_call(
    _layer_body,
    out_shape=jax.ShapeDtypeStruct((NP, C), jnp.float32),
)


def _out_body(h_ref, w_ref, b_ref, o_ref):
    logits = jnp.dot(h_ref[:N], w_ref[...],
                     preferred_element_type=jnp.float32) + b_ref[...]
    m = jnp.max(logits, axis=1, keepdims=True)
    z = logits - m
    o_ref[...] = z - jnp.log(jnp.sum(jnp.exp(z), axis=1, keepdims=True))


_tc_out = pl.pallas_call(
    _out_body,
    out_shape=jax.ShapeDtypeStruct((N, NCLS), jnp.float32),
)


def kernel(x, edge_index, edge_index_global, edge_weight_global,
           W_in, b_in, W_layers, b_layers, W_out, b_out, alpha, gamma):
    pad = EP - E
    srclp = jnp.pad(edge_index[0], (0, pad))
    dstlp = jnp.pad(edge_index[1], (0, pad),
                    constant_values=NP - 1).reshape(EPC, CH)
    srcgp = jnp.pad(edge_index_global[0], (0, pad))
    dstgp = jnp.pad(edge_index_global[1], (0, pad),
                    constant_values=NP - 1).reshape(EPC, CH)

    degp = _sc_degree(dstlp)

    xp = jnp.pad(x, ((0, NP - N), (0, 0)))
    scal = jnp.stack([alpha, gamma]).reshape(1, 2)
    deg2d = degp.reshape(NC, NP // 128, 128)
    ew2d = edge_weight_global.reshape(ECHUNKS, 128)
    h0, ax0, invd2d, wg2d = _tc_pre(xp, W_in, b_in.reshape(1, C), deg2d,
                                    ew2d, scal)
    invd = invd2d.reshape(NP)
    wgp = jnp.pad(wg2d, ((0, EPC - ECHUNKS), (0, 0)))

    h = h0
    for i in range(L - 2):
        part = _sc_aggregate(h, srclp, dstlp, srcgp, dstgp, wgp, invd)
        h = _tc_layer(part, ax0, W_layers[i], b_layers[i].reshape(1, C))

    return _tc_out(h, W_out, b_out.reshape(1, NCLS))


# speedup vs baseline: 1.2224x; 1.2224x over previous
import functools

import jax
import jax.numpy as jnp
from jax import lax
from jax.experimental import pallas as pl
from jax.experimental.pallas import tpu as pltpu
from jax.experimental.pallas import tpu_sc as plsc

N = 10000
E = 320000
D = 128
C = 64
L = 24
NCLS = 10

NC = 2
NS = 16
NW = NC * NS
LN = 16

NP = 10240
RPW = NP // NW
CH = 128
ECHUNKS = E // CH
EPC = 2560
EPD = EPC * CH
NCH_W = EPC // NW
DUMP = 370
ACCR = 384

BLK = 2048
NBLK = 157
EPS = NBLK * BLK
SENT = 0x7FFF0000
CAPW = 12288
CAPW4 = CAPW * 4

NB = 4
LA = 2

_mesh = plsc.VectorSubcoreMesh(
    core_axis_name="c", subcore_axis_name="s", num_cores=NC, num_subcores=NS)

_SC_PARAMS = pltpu.CompilerParams(use_tc_tiling_on_sc=False,
                                  needs_layout_passes=False)


@functools.partial(
    pl.kernel,
    out_type=jax.ShapeDtypeStruct((NC, NP), jnp.float32),
    mesh=_mesh,
    scratch_types=[
        pltpu.VMEM_SHARED((NP,), jnp.float32),
        pltpu.VMEM((NCH_W, CH), jnp.int32),
        pltpu.VMEM((CH,), jnp.float32),
        pltpu.VMEM((NP // NS,), jnp.float32),
    ],
    compiler_params=_SC_PARAMS,
)
def _sc_degree(dst_hbm, out_hbm, acc, didx, ones_v, slice_v):
    cid = lax.axis_index("c")
    sid = lax.axis_index("s")
    wid = sid * NC + cid
    rpt = NP // NS
    base = sid * rpt

    def _z(i, carry):
        slice_v[pl.ds(i * LN, LN)] = jnp.zeros((LN,), jnp.float32)
        return carry
    lax.fori_loop(0, rpt // LN, _z, 0)
    pltpu.sync_copy(slice_v, acc.at[pl.ds(base, rpt)])

    def _o(i, carry):
        ones_v[pl.ds(i * LN, LN)] = jnp.ones((LN,), jnp.float32)
        return carry
    lax.fori_loop(0, CH // LN, _o, 0)
    pltpu.sync_copy(dst_hbm.at[pl.ds(wid * NCH_W, NCH_W)], didx)
    plsc.subcore_barrier()

    def _body(i, carry):
        pltpu.sync_copy(ones_v, acc.at[didx.at[i]], add=True)
        return carry
    lax.fori_loop(0, NCH_W, _body, 0)
    plsc.subcore_barrier()

    pltpu.sync_copy(acc.at[pl.ds(base, rpt)], slice_v)
    pltpu.sync_copy(slice_v, out_hbm.at[cid, pl.ds(base, rpt)])


@functools.partial(
    pl.kernel,
    out_type=(
        jax.ShapeDtypeStruct((NW, CAPW4), jnp.int32),
        jax.ShapeDtypeStruct((NW, CAPW4), jnp.int32),
        jax.ShapeDtypeStruct((2, NW, 16), jnp.int32),
    ),
    mesh=_mesh,
    scratch_types=[
        pltpu.VMEM((2, BLK), jnp.int32),
        pltpu.VMEM((2, BLK), jnp.int32),
        pltpu.VMEM((2, BLK), jnp.int32),
        pltpu.VMEM((CAPW4,), jnp.int32),
        pltpu.VMEM((16,), jnp.int32),
        pltpu.SemaphoreType.DMA((2,)),
    ],
    compiler_params=_SC_PARAMS,
)
def _sc_bucket(srcl, dstl, srcg, dstg, wgi,
               bl_out, bg_out, cnt_out, sblk, dblk, wblk, staging, cbuf,
               bsem):
    cid = lax.axis_index("c")
    sid = lax.axis_index("s")
    wid = sid * NC + cid
    lo = wid * RPW
    hi = lo + RPW
    iota = lax.iota(jnp.int32, LN)
    dump_pat = jnp.where(iota % 4 == 1, jnp.int32(DUMP), jnp.int32(0))

    def _issue(sref, dref, wref, p, b, weighted):
        off = b * BLK
        pltpu.async_copy(sref.at[pl.ds(off, BLK)], sblk.at[p], bsem.at[p])
        pltpu.async_copy(dref.at[pl.ds(off, BLK)], dblk.at[p], bsem.at[p])
        if weighted:
            pltpu.async_copy(wref.at[pl.ds(off, BLK)], wblk.at[p],
                             bsem.at[p])

    def _scan_set(sref, dref, wref, bout, set_idx, weighted):
        def _fillstage(t, carry):
            staging[pl.ds(t * LN, LN)] = dump_pat
            return carry
        lax.fori_loop(0, CAPW4 // LN, _fillstage, 0)

        for p in range(2):
            _issue(sref, dref, wref, p, p, weighted)

        def _grp(g, wptr):
            for p in range(2):
                b = g * 2 + p

                @pl.when(b < NBLK)
                def _w():
                    for _ in range(3 if weighted else 2):
                        pltpu.make_async_copy(
                            sref.at[pl.ds(0, BLK)], sblk.at[p],
                            bsem.at[p]).wait()

                valid = b < NBLK

                def _vec(kk, wp):
                    d16 = dblk[p, pl.ds(kk * LN, LN)]
                    mask = jnp.logical_and(
                        jnp.logical_and(d16 >= lo, d16 < hi), valid)
                    mi = mask.astype(jnp.int32)
                    cnt = plsc.all_reduce_population_count(mask)[0]
                    ranks = plsc.cumsum(mi) - mi
                    wp2 = jnp.minimum(wp, CAPW - LN)
                    pos = (wp2 + ranks) * 4
                    s16 = sblk[p, pl.ds(kk * LN, LN)]
                    plsc.store_scatter(staging, [pos], s16, mask=mask)
                    plsc.store_scatter(staging, [pos + 1], d16 - lo,
                                       mask=mask)
                    if weighted:
                        w16 = wblk[p, pl.ds(kk * LN, LN)]
                        plsc.store_scatter(staging, [pos + 2], w16,
                                           mask=mask)
                    return wp + cnt
                wptr2 = lax.fori_loop(0, BLK // LN, _vec, wptr)

                @pl.when(b + 2 < NBLK)
                def _i():
                    _issue(sref, dref, wref, p, b + 2, weighted)
                wptr = wptr2
            return wptr
        wptr = lax.fori_loop(0, (NBLK + 1) // 2, _grp, 0)

        wptr = jnp.minimum(wptr, CAPW)
        cbuf[pl.ds(0, LN)] = jnp.zeros((LN,), jnp.int32) + wptr
        pltpu.sync_copy(cbuf, cnt_out.at[set_idx, wid])
        pltpu.sync_copy(staging, bout.at[wid])

    _scan_set(srcl, dstl, wgi, bl_out, 0, weighted=False)
    _scan_set(srcg, dstg, wgi, bg_out, 1, weighted=True)


@functools.partial(
    pl.kernel,
    out_type=jax.ShapeDtypeStruct((NP, C), jnp.float32),
    mesh=_mesh,
    scratch_types=[
        pltpu.VMEM((ACCR, C), jnp.float32),
        pltpu.VMEM((CAPW4,), jnp.int32),
        pltpu.VMEM((NB, CH, C), jnp.float32),
        pltpu.VMEM((NB, CH), jnp.int32),
        pltpu.VMEM((RPW,), jnp.float32),
        pltpu.VMEM((16,), jnp.float32),
        pltpu.VMEM((16,), jnp.int32),
        pltpu.SemaphoreType.DMA((NB,)),
    ],
    compiler_params=_SC_PARAMS,
)
def _sc_aggregate(h_hbm, bl_hbm, bg_hbm, cnt_hbm, invd_hbm,
                  out_hbm, acc, recb, rows, sidx, invdv, cgb, cntb, gsem):
    cid = lax.axis_index("c")
    sid = lax.axis_index("s")
    wid = sid * NC + cid
    iota = lax.iota(jnp.int32, LN)
    iota4 = iota * 4
    cols = [iota + j * LN for j in range(C // LN)]

    def _z(r, carry):
        for j in range(C // LN):
            acc[r, pl.ds(j * LN, LN)] = jnp.zeros((LN,), jnp.float32)
        return carry
    lax.fori_loop(0, ACCR, _z, 0)

    pltpu.sync_copy(invd_hbm.at[pl.ds(wid * RPW, RPW)], invdv)
    pltpu.sync_copy(invd_hbm.at[pl.ds(NP - LN, LN)], cgb)
    cg1 = cgb[pl.ds(0, LN)][LN - 1]

    def _bucket_pass(bref, set_idx, weighted):
        pltpu.sync_copy(cnt_hbm.at[set_idx, wid], cntb)
        cnt = cntb[pl.ds(0, LN)][0]
        nch = (cnt + CH - 1) // CH
        pltpu.sync_copy(bref.at[wid], recb)
        ngroups = (nch + LA + NB - 1) // NB

        def _grp(g, carry):
            for u in range(NB):
                i = g * NB + u

                @pl.when(i < nch)
                def _issue():
                    def _ex(kk, c2):
                        base4 = (i * CH + kk * LN) * 4
                        s16 = plsc.load_gather(recb, [base4 + iota4])
                        s16 = jnp.clip(s16, 0, N - 1)
                        sidx[u, pl.ds(kk * LN, LN)] = s16
                        return c2
                    lax.fori_loop(0, CH // LN, _ex, 0)
                    pltpu.async_copy(h_hbm.at[sidx.at[u]], rows.at[u],
                                     gsem.at[u])

                k = i - LA
                bu = (u + LA) % NB

                @pl.when(jnp.logical_and(k >= 0, k < nch))
                def _process():
                    pltpu.make_async_copy(
                        h_hbm.at[sidx.at[bu]], rows.at[bu],
                        gsem.at[bu]).wait()

                    def _proc(kk, c3):
                        base4 = (k * CH + kk * LN) * 4
                        dl16 = jnp.clip(
                            plsc.load_gather(recb, [base4 + 1 + iota4]),
                            0, ACCR - 1)
                        if weighted:
                            w16 = plsc.bitcast(
                                plsc.load_gather(recb, [base4 + 2 + iota4]),
                                jnp.float32) * cg1
                        for ii in range(LN):
                            ridx = jnp.zeros((LN,), jnp.int32) + dl16[ii]
                            r = kk * LN + ii
                            for j in range(C // LN):
                                v = rows[bu, r, pl.ds(j * LN, LN)]
                                if weighted:
                                    v = v * w16[ii]
                                plsc.addupdate_scatter(
                                    acc, [ridx, cols[j]], v)
                        return c3
                    lax.fori_loop(0, CH // LN, _proc, 0)
            return carry
        lax.fori_loop(0, ngroups, _grp, 0)

    _bucket_pass(bl_hbm, 0, weighted=False)

    def _scale(r16, carry):
        s16 = invdv[pl.ds(r16 * LN, LN)]
        for ii in range(LN):
            s = s16[ii]
            r = r16 * LN + ii
            for j in range(C // LN):
                acc[r, pl.ds(j * LN, LN)] = acc[r, pl.ds(j * LN, LN)] * s
        return carry
    lax.fori_loop(0, RPW // LN, _scale, 0)

    _bucket_pass(bg_hbm, 1, weighted=True)

    pltpu.sync_copy(acc.at[pl.ds(0, RPW)],
                    out_hbm.at[pl.ds(wid * RPW, RPW)])


def _pre_body(x_ref, w_ref, b_ref, deg_ref, sc_ref,
              h0_ref, ax0_ref, invd_ref):
    alpha = sc_ref[0, 0]
    gamma = sc_ref[0, 1]
    a1 = 1.0 - alpha
    h0 = jnp.dot(x_ref[...], w_ref[...],
                 preferred_element_type=jnp.float32) + b_ref[...]
    h0_ref[...] = h0
    ax0_ref[...] = alpha * h0
    d = deg_ref[0] + deg_ref[1]
    inv = a1 / jnp.maximum(d, 1.0)
    ri = lax.broadcasted_iota(jnp.int32, inv.shape, 0)
    ci = lax.broadcasted_iota(jnp.int32, inv.shape, 1)
    islast = jnp.logical_and(ri == inv.shape[0] - 1, ci == inv.shape[1] - 1)
    invd_ref[...] = jnp.where(islast, a1 * gamma, inv)


_tc_pre = pl.pallas_call(
    _pre_body,
    out_shape=(
        jax.ShapeDtypeStruct((NP, C), jnp.float32),
        jax.ShapeDtypeStruct((NP, C), jnp.float32),
        jax.ShapeDtypeStruct((NP // 128, 128), jnp.float32),
    ),
    in_specs=[
        pl.BlockSpec(memory_space=pltpu.VMEM),
        pl.BlockSpec(memory_space=pltpu.VMEM),
        pl.BlockSpec(memory_space=pltpu.VMEM),
        pl.BlockSpec(memory_space=pltpu.VMEM),
        pl.BlockSpec(memory_space=pltpu.SMEM),
    ],
)


def _layer_body(p_ref, ax0_ref, w_ref, b_ref, h_ref):
    hp = p_ref[...] + ax0_ref[...]
    h = jnp.dot(hp, w_ref[...], preferred_element_type=jnp.float32) + b_ref[...]
    h_ref[...] = jnp.maximum(h, 0.0)


_tc_layer = pl.pallas_call(
    _layer_body,
    out_shape=jax.ShapeDtypeStruct((NP, C), jnp.float32),
)


def _out_body(h_ref, w_ref, b_ref, o_ref):
    logits = jnp.dot(h_ref[:N], w_ref[...],
                     preferred_element_type=jnp.float32) + b_ref[...]
    m = jnp.max(logits, axis=1, keepdims=True)
    z = logits - m
    o_ref[...] = z - jnp.log(jnp.sum(jnp.exp(z), axis=1, keepdims=True))


_tc_out = pl.pallas_call(
    _out_body,
    out_shape=jax.ShapeDtypeStruct((N, NCLS), jnp.float32),
)


def kernel(x, edge_index, edge_index_global, edge_weight_global,
           W_in, b_in, W_layers, b_layers, W_out, b_out, alpha, gamma):
    dstl_deg = jnp.pad(edge_index[1], (0, EPD - E),
                       constant_values=NP - 1).reshape(EPC, CH)
    degp = _sc_degree(dstl_deg)

    pad = EPS - E
    sl2 = jnp.pad(edge_index[0], (0, pad))
    dl2 = jnp.pad(edge_index[1], (0, pad), constant_values=SENT)
    sg2 = jnp.pad(edge_index_global[0], (0, pad))
    dg2 = jnp.pad(edge_index_global[1], (0, pad), constant_values=SENT)
    wgi = lax.bitcast_convert_type(
        jnp.pad(edge_weight_global, (0, pad)), jnp.int32)
    bl, bg, counts = _sc_bucket(sl2, dl2, sg2, dg2, wgi)

    xp = jnp.pad(x, ((0, NP - N), (0, 0)))
    scal = jnp.stack([alpha, gamma]).reshape(1, 2)
    deg2d = degp.reshape(NC, NP // 128, 128)
    h0, ax0, invd2d = _tc_pre(xp, W_in, b_in.reshape(1, C), deg2d, scal)
    invd = invd2d.reshape(NP)

    h = h0
    for i in range(L - 2):
        part = _sc_aggregate(h, bl, bg, counts, invd)
        h = _tc_layer(part, ax0, W_layers[i], b_layers[i].reshape(1, C))

    return _tc_out(h, W_out, b_out.reshape(1, NCLS))
